# baseline (speedup 1.0000x reference)
import functools

import jax
import jax.numpy as jnp
import numpy as np
from jax import lax
from jax.experimental import pallas as pl
from jax.experimental.pallas import tpu as pltpu
from jax.experimental.pallas import tpu_sc as plsc

NC = 2
NS = 16
NW = NC * NS
LANES = 16

ROWS = 128
COLS = 32768
ROWS_PER_W = ROWS // NW

ACC = 4
UNROLL = 8
CHUNK = 8192
NBUF = 4
CHUNKS_PER_ROW = COLS // CHUNK
NCHUNKS = ROWS_PER_W * CHUNKS_PER_ROW
VREGS_PER_STEP = ACC * UNROLL
STEPS_PER_CHUNK = CHUNK // (VREGS_PER_STEP * LANES)

_INT_MAX = np.int32(2147483647)


@functools.partial(
    pl.kernel,
    mesh=plsc.VectorSubcoreMesh(core_axis_name="c", subcore_axis_name="s"),
    out_type=jax.ShapeDtypeStruct((NW, LANES), jnp.int32),
    compiler_params=pltpu.CompilerParams(needs_layout_passes=False),
    scratch_types=[
        pltpu.VMEM((CHUNK,), jnp.float32),
        pltpu.VMEM((CHUNK,), jnp.float32),
        pltpu.VMEM((CHUNK,), jnp.float32),
        pltpu.VMEM((CHUNK,), jnp.float32),
        pltpu.VMEM((LANES,), jnp.int32),
        pltpu.SemaphoreType.DMA,
        pltpu.SemaphoreType.DMA,
        pltpu.SemaphoreType.DMA,
        pltpu.SemaphoreType.DMA,
    ],
)
def _argmax_sc(x_hbm, out_hbm, buf0, buf1, buf2, buf3, res_ref,
               sem0, sem1, sem2, sem3):
    bufs = (buf0, buf1, buf2, buf3)
    wid = lax.axis_index("s") * NC + lax.axis_index("c")
    base_row = wid * ROWS_PER_W
    sems = (sem0, sem1, sem2, sem3)
    lane = lax.iota(jnp.int32, LANES)

    def chunk_src(c):
        row = c // CHUNKS_PER_ROW
        part = c % CHUNKS_PER_ROW
        return x_hbm.at[base_row + row, pl.ds(part * CHUNK, CHUNK)]

    copies = [None] * NCHUNKS
    for c in range(NBUF - 1):
        copies[c] = pltpu.async_copy(chunk_src(c), bufs[c % NBUF], sems[c % NBUF])

    res = jnp.zeros((LANES,), jnp.int32)
    neg_inf = jnp.full((LANES,), -jnp.inf, jnp.float32)
    zero_i = jnp.zeros((LANES,), jnp.int32)

    for j in range(ROWS_PER_W):
        best = [neg_inf] * ACC
        bidx = [zero_i] * ACC
        idx = [lane + a * LANES for a in range(ACC)]

        for p in range(CHUNKS_PER_ROW):
            c = j * CHUNKS_PER_ROW + p
            if c + NBUF - 1 < NCHUNKS:
                cn = c + NBUF - 1
                copies[cn] = pltpu.async_copy(
                    chunk_src(cn), bufs[cn % NBUF], sems[cn % NBUF]
                )
            copies[c].wait()
            buf = bufs[c % NBUF]

            def step(i, carry, buf=buf):
                best, bidx, idx = list(carry[0]), list(carry[1]), list(carry[2])
                for u in range(UNROLL):
                    for a in range(ACC):
                        k = i * VREGS_PER_STEP + u * ACC + a
                        v = buf[pl.ds(k * LANES, LANES)]
                        m = v > best[a]
                        best[a] = jnp.where(m, v, best[a])
                        bidx[a] = jnp.where(m, idx[a], bidx[a])
                        idx[a] = idx[a] + ACC * LANES
                return tuple(best), tuple(bidx), tuple(idx)

            carry = lax.fori_loop(
                0, STEPS_PER_CHUNK, step, (tuple(best), tuple(bidx), tuple(idx))
            )
            best, bidx, idx = list(carry[0]), list(carry[1]), list(carry[2])

        def merge(b1, i1, b2, i2):
            m = (b2 > b1) | ((b2 == b1) & (i2 < i1))
            return jnp.where(m, b2, b1), jnp.where(m, i2, i1)

        b01, i01 = merge(best[0], bidx[0], best[1], bidx[1])
        b23, i23 = merge(best[2], bidx[2], best[3], bidx[3])
        ball, iall = merge(b01, i01, b23, i23)

        row_max = jnp.max(ball)
        cand = jnp.where(ball == row_max, iall, _INT_MAX)
        row_arg = jnp.min(cand)
        res = jnp.where(lane == j, row_arg, res)

    res_ref[...] = res
    pltpu.sync_copy(res_ref, out_hbm.at[wid])


def kernel(x):
    out = _argmax_sc(x)
    return out[:, :ROWS_PER_W].reshape(ROWS).astype(jnp.int64)


# speedup vs baseline: 1.2069x; 1.2069x over previous
import functools

import jax
import jax.numpy as jnp
import numpy as np
from jax import lax
from jax.experimental import pallas as pl
from jax.experimental.pallas import tpu as pltpu
from jax.experimental.pallas import tpu_sc as plsc

NC = 2
NS = 16
NW = NC * NS
LANES = 16

ROWS = 128
COLS = 32768
ROWS_PER_W = ROWS // NW

ACC = 4
UNROLL = 4
CHUNK = 16384
VREGS_PER_STEP = ACC * UNROLL
STEPS_PER_CHUNK = CHUNK // (VREGS_PER_STEP * LANES)

_INT_MAX = np.int32(2147483647)


@functools.partial(
    pl.kernel,
    mesh=plsc.VectorSubcoreMesh(core_axis_name="c", subcore_axis_name="s"),
    out_type=jax.ShapeDtypeStruct((NW, LANES), jnp.int32),
    compiler_params=pltpu.CompilerParams(needs_layout_passes=False),
    scratch_types=[
        pltpu.VMEM((CHUNK,), jnp.float32),
        pltpu.VMEM((CHUNK,), jnp.float32),
        pltpu.VMEM((LANES,), jnp.int32),
        pltpu.SemaphoreType.DMA,
        pltpu.SemaphoreType.DMA,
    ],
)
def _argmax_sc(x_hbm, out_hbm, buf0, buf1, res_ref, sem0, sem1):
    bufs = (buf0, buf1)
    sems = (sem0, sem1)
    wid = lax.axis_index("s") * NC + lax.axis_index("c")
    base_row = wid * ROWS_PER_W
    lane = lax.iota(jnp.int32, LANES)

    def chunk_src(c):
        return x_hbm.at[base_row + c // 2, pl.ds((c % 2) * CHUNK, CHUNK)]

    pltpu.async_copy(chunk_src(0), buf0, sem0)
    pltpu.async_copy(chunk_src(1), buf1, sem1)

    neg_inf = jnp.full((LANES,), -jnp.inf, jnp.float32)
    zero_i = jnp.zeros((LANES,), jnp.int32)

    def row_body(j, res):
        best = [neg_inf] * ACC
        bidx = [zero_i] * ACC
        idx = [lane + a * LANES for a in range(ACC)]

        for half in range(2):
            c = 2 * j + half
            buf = bufs[half]
            pltpu.make_async_copy(chunk_src(c), buf, sems[half]).wait()

            def step(i, carry, buf=buf):
                best, bidx, idx = list(carry[0]), list(carry[1]), list(carry[2])
                for u in range(UNROLL):
                    for a in range(ACC):
                        k = i * VREGS_PER_STEP + u * ACC + a
                        v = buf[pl.ds(k * LANES, LANES)]
                        m = v > best[a]
                        best[a] = jnp.where(m, v, best[a])
                        bidx[a] = jnp.where(m, idx[a], bidx[a])
                        idx[a] = idx[a] + ACC * LANES
                return tuple(best), tuple(bidx), tuple(idx)

            carry = lax.fori_loop(
                0, STEPS_PER_CHUNK, step, (tuple(best), tuple(bidx), tuple(idx))
            )
            best, bidx, idx = list(carry[0]), list(carry[1]), list(carry[2])

            @pl.when(j < ROWS_PER_W - 1)
            def _issue(c=c, half=half):
                pltpu.async_copy(chunk_src(c + 2), bufs[half], sems[half])

        def merge(b1, i1, b2, i2):
            m = (b2 > b1) | ((b2 == b1) & (i2 < i1))
            return jnp.where(m, b2, b1), jnp.where(m, i2, i1)

        b01, i01 = merge(best[0], bidx[0], best[1], bidx[1])
        b23, i23 = merge(best[2], bidx[2], best[3], bidx[3])
        ball, iall = merge(b01, i01, b23, i23)

        row_max = jnp.max(ball)
        cand = jnp.where(ball == row_max, iall, _INT_MAX)
        row_arg = jnp.min(cand)
        return jnp.where(lane == j, row_arg, res)

    res = lax.fori_loop(0, ROWS_PER_W, row_body, jnp.zeros((LANES,), jnp.int32))

    res_ref[...] = res
    pltpu.sync_copy(res_ref, out_hbm.at[wid])


def kernel(x):
    out = _argmax_sc(x)
    return out[:, :ROWS_PER_W].reshape(ROWS).astype(jnp.int64)
